# baseline (speedup 1.0000x reference)
import jax
import jax.numpy as jnp
import numpy as np
from jax.experimental import pallas as pl
from jax.experimental.pallas import tpu as pltpu

STATE_DIM = 256
ACTION_DIM = 64
CEM_BATCH = 64
CEM_TOPK = 8
CEM_ITERATIONS = 3
HIDDEN = 512

BB = 128

_CONSTS = {}


def _consts(B):
    if B not in _CONSTS:
        with jax.ensure_compile_time_eval():
            @jax.jit
            def _build():
                base = jax.random.key(42)
                eps = [
                    jax.random.normal(jax.random.fold_in(base, i),
                                      (CEM_BATCH, B, ACTION_DIM),
                                      dtype=jnp.float32)
                    for i in range(CEM_ITERATIONS)
                ]
                return jnp.tanh(eps[0]), jnp.stack(eps[1:])
            t0, rest = _build()
            _CONSTS[B] = (np.asarray(t0), np.asarray(rest))
    return _CONSTS[B]


def _argmax_gather(q, actions, cand_iota):
    m = jnp.max(q, axis=0, keepdims=True)
    idx = jnp.min(jnp.where(q == m, cand_iota, CEM_BATCH),
                  axis=0, keepdims=True)
    onehot = (cand_iota == idx).astype(jnp.float32)
    a_sel = jnp.sum(actions * onehot[:, :, None], axis=0)
    return a_sel, onehot


def _cem_kernel(state_ref, w1s_ref, w1a_ref, b1_ref, w2p_ref, t0_ref, eps_ref,
                out_ref):
    h_state = jnp.dot(state_ref[...], w1s_ref[...],
                      preferred_element_type=jnp.float32) + b1_ref[...]

    cand_iota = jax.lax.broadcasted_iota(jnp.int32, (CEM_BATCH, BB), 0)
    w1a = w1a_ref[...]
    w2p = w2p_ref[...]
    lane0 = (jax.lax.broadcasted_iota(jnp.int32, (1, 1, 128), 2)
             == 0).astype(jnp.float32)

    def q_of(actions):
        a2d = actions.reshape(CEM_BATCH * BB, ACTION_DIM)
        h = jnp.dot(a2d, w1a, preferred_element_type=jnp.float32)
        h = h.reshape(CEM_BATCH, BB, HIDDEN) + h_state[None]
        h = jnp.maximum(h, 0.0)
        qw = jnp.dot(h.reshape(CEM_BATCH * BB, HIDDEN), w2p,
                     preferred_element_type=jnp.float32)
        qw = qw.reshape(CEM_BATCH, BB, 128)
        return jnp.sum(qw * lane0, axis=2)

    actions = t0_ref[...]
    for i in range(CEM_ITERATIONS):
        q = q_of(actions)
        if i == CEM_ITERATIONS - 1:
            best, _ = _argmax_gather(q, actions, cand_iota)
            out_ref[...] = best
            return
        sel = []
        for _ in range(CEM_TOPK):
            a_sel, onehot = _argmax_gather(q, actions, cand_iota)
            sel.append(a_sel)
            q = jnp.where(onehot > 0, -jnp.inf, q)
        atk = jnp.stack(sel)
        mean = jnp.mean(atk, axis=0)
        std = jnp.std(atk, axis=0, ddof=1)
        actions = jnp.tanh(mean[None] + std[None] * eps_ref[i])


def kernel(state, W1, b1, W2, b2):
    B = state.shape[0]
    t0, eps_rest = _consts(B)
    w1s = W1[:STATE_DIM]
    w1a = W1[STATE_DIM:]
    b1r = b1.reshape(1, HIDDEN)
    w2p = jnp.concatenate([W2, jnp.zeros((HIDDEN, 127), jnp.float32)], axis=1)
    grid = B // BB
    return pl.pallas_call(
        _cem_kernel,
        grid=(grid,),
        in_specs=[
            pl.BlockSpec((BB, STATE_DIM), lambda i: (i, 0)),
            pl.BlockSpec((STATE_DIM, HIDDEN), lambda i: (0, 0)),
            pl.BlockSpec((ACTION_DIM, HIDDEN), lambda i: (0, 0)),
            pl.BlockSpec((1, HIDDEN), lambda i: (0, 0)),
            pl.BlockSpec((HIDDEN, 128), lambda i: (0, 0)),
            pl.BlockSpec((CEM_BATCH, BB, ACTION_DIM), lambda i: (0, i, 0)),
            pl.BlockSpec((CEM_ITERATIONS - 1, CEM_BATCH, BB, ACTION_DIM),
                         lambda i: (0, 0, i, 0)),
        ],
        out_specs=pl.BlockSpec((BB, ACTION_DIM), lambda i: (i, 0)),
        out_shape=jax.ShapeDtypeStruct((B, ACTION_DIM), jnp.float32),
        compiler_params=pltpu.CompilerParams(
            dimension_semantics=("parallel",)),
    )(state, w1s, w1a, b1r, w2p, jnp.asarray(t0), jnp.asarray(eps_rest))


# speedup vs baseline: 1.1317x; 1.1317x over previous
import jax
import jax.numpy as jnp
import numpy as np
from jax.experimental import pallas as pl
from jax.experimental.pallas import tpu as pltpu

STATE_DIM = 256
ACTION_DIM = 64
CEM_BATCH = 64
CEM_TOPK = 8
CEM_ITERATIONS = 3
HIDDEN = 512

BB = 128

_CONSTS = {}


def _consts(B):
    if B not in _CONSTS:
        with jax.ensure_compile_time_eval():
            @jax.jit
            def _build():
                base = jax.random.key(42)
                eps = [
                    jax.random.normal(jax.random.fold_in(base, i),
                                      (CEM_BATCH, B, ACTION_DIM),
                                      dtype=jnp.float32)
                    for i in range(CEM_ITERATIONS)
                ]
                return jnp.tanh(eps[0]), jnp.stack(eps[1:])
            t0, rest = _build()
            _CONSTS[B] = (np.asarray(t0), np.asarray(rest))
    return _CONSTS[B]


def _argmax_gather(q, actions, cand_iota):
    m = jnp.max(q, axis=0, keepdims=True)
    idx = jnp.min(jnp.where(q == m, cand_iota, CEM_BATCH),
                  axis=0, keepdims=True)
    onehot = (cand_iota == idx).astype(jnp.float32)
    a_sel = jnp.sum(actions * onehot[:, :, None], axis=0)
    return a_sel, onehot


def _cem_kernel(state_ref, w1s_ref, w1a_ref, b1_ref, w2_ref, t0_ref, eps_ref,
                out_ref):
    h_state = jnp.dot(state_ref[...], w1s_ref[...],
                      preferred_element_type=jnp.float32) + b1_ref[...]

    cand_iota = jax.lax.broadcasted_iota(jnp.int32, (CEM_BATCH, BB), 0)
    w1a = w1a_ref[...]
    w2 = w2_ref[...].astype(jnp.bfloat16).astype(jnp.float32)
    w2 = w2.reshape(1, 1, HIDDEN)

    def q_of(actions):
        a2d = actions.reshape(CEM_BATCH * BB, ACTION_DIM)
        h = jnp.dot(a2d, w1a, preferred_element_type=jnp.float32)
        h = h.reshape(CEM_BATCH, BB, HIDDEN) + h_state[None]
        h = jnp.maximum(h, 0.0)
        hb = h.astype(jnp.bfloat16).astype(jnp.float32)
        return jnp.sum(hb * w2, axis=2)

    actions = t0_ref[...]
    for i in range(CEM_ITERATIONS):
        q = q_of(actions)
        if i == CEM_ITERATIONS - 1:
            best, _ = _argmax_gather(q, actions, cand_iota)
            out_ref[...] = best
            return
        sel = []
        for _ in range(CEM_TOPK):
            a_sel, onehot = _argmax_gather(q, actions, cand_iota)
            sel.append(a_sel)
            q = jnp.where(onehot > 0, -jnp.inf, q)
        atk = jnp.stack(sel)
        mean = jnp.mean(atk, axis=0)
        std = jnp.std(atk, axis=0, ddof=1)
        actions = jnp.tanh(mean[None] + std[None] * eps_ref[i])


def kernel(state, W1, b1, W2, b2):
    B = state.shape[0]
    t0, eps_rest = _consts(B)
    w1s = W1[:STATE_DIM]
    w1a = W1[STATE_DIM:]
    b1r = b1.reshape(1, HIDDEN)
    w2r = W2.reshape(1, HIDDEN)
    grid = B // BB
    return pl.pallas_call(
        _cem_kernel,
        grid=(grid,),
        in_specs=[
            pl.BlockSpec((BB, STATE_DIM), lambda i: (i, 0)),
            pl.BlockSpec((STATE_DIM, HIDDEN), lambda i: (0, 0)),
            pl.BlockSpec((ACTION_DIM, HIDDEN), lambda i: (0, 0)),
            pl.BlockSpec((1, HIDDEN), lambda i: (0, 0)),
            pl.BlockSpec((1, HIDDEN), lambda i: (0, 0)),
            pl.BlockSpec((CEM_BATCH, BB, ACTION_DIM), lambda i: (0, i, 0)),
            pl.BlockSpec((CEM_ITERATIONS - 1, CEM_BATCH, BB, ACTION_DIM),
                         lambda i: (0, 0, i, 0)),
        ],
        out_specs=pl.BlockSpec((BB, ACTION_DIM), lambda i: (i, 0)),
        out_shape=jax.ShapeDtypeStruct((B, ACTION_DIM), jnp.float32),
        compiler_params=pltpu.CompilerParams(
            dimension_semantics=("parallel",)),
    )(state, w1s, w1a, b1r, w2r, jnp.asarray(t0), jnp.asarray(eps_rest))
